# baseline (speedup 1.0000x reference)
import functools

import jax
import jax.numpy as jnp
from jax import lax
from jax.experimental import pallas as pl
from jax.experimental.pallas import tpu as pltpu
from jax.experimental.pallas import tpu_sc as plsc

N = 10000
G = 64
F_IN = 256
H = 256
OUT3 = 32

NC = 2
NS = 16
GROUP = 64
NBUF = 4
DNBUF = 4
E_PAD = 163840
NGRP = E_PAD // GROUP
GPT = NGRP // NS
GPW = NGRP // (NC * NS)
HGRP = GPT // 4
NROWS = 10112
RPT = NROWS // NS

BN = 400
NBLK = N // BN


def _deg_body(dst_hbm, ones_hbm, zeros_hbm, out_hbm, dstv, onesv,
              acc, s0, s1, s2, s3):
    c = lax.axis_index("c")
    s = lax.axis_index("s")
    w = c * NS + s
    sems = [s0, s1, s2, s3]
    r0 = pl.multiple_of(s * RPT, 8)
    pltpu.sync_copy(dst_hbm.at[pl.ds(w * GPW, GPW)], dstv)
    pltpu.sync_copy(ones_hbm, onesv)
    pltpu.sync_copy(zeros_hbm, acc.at[pl.ds(r0, RPT)])
    plsc.subcore_barrier()

    def step(j, carry):
        for b in range(DNBUF):
            g = j * DNBUF + b

            @pl.when(j > 0)
            def _():
                pltpu.make_async_copy(
                    onesv, acc.at[dstv.at[g - DNBUF]], sems[b]).wait()

            pltpu.async_copy(onesv, acc.at[dstv.at[g]], sems[b], add=True)
        return carry

    lax.fori_loop(0, GPW // DNBUF, step, 0)
    for b in range(DNBUF):
        g = GPW - DNBUF + b
        pltpu.make_async_copy(onesv, acc.at[dstv.at[g]], sems[b]).wait()
    plsc.subcore_barrier()
    pltpu.sync_copy(acc.at[pl.ds(r0, RPT)], out_hbm.at[c, pl.ds(r0, RPT)])


@functools.cache
def _deg_call():
    return pl.kernel(
        _deg_body,
        out_type=jax.ShapeDtypeStruct((NC, NROWS, 128), jnp.float32),
        mesh=plsc.VectorSubcoreMesh(core_axis_name="c", subcore_axis_name="s"),
        scratch_types=[
            pltpu.VMEM((GPW, GROUP), jnp.int32),
            pltpu.VMEM((GROUP, 128), jnp.float32),
            pltpu.VMEM_SHARED((NROWS, 128), jnp.float32),
        ] + [pltpu.SemaphoreType.DMA] * DNBUF,
    )


def _make_agg_body(d):
    gpt = GPT
    nh = gpt // HGRP

    def body(y_hbm, idx_hbm, zeros_hbm, out_hbm, ibv, *bufs):
        rbs_ = list(bufs[:NBUF])
        acc = bufs[NBUF]
        gsems_ = list(bufs[NBUF + 1:NBUF + 1 + NBUF])
        ssems_ = list(bufs[NBUF + 1 + NBUF:])
        c = lax.axis_index("c")
        s = lax.axis_index("s")
        rbs = rbs_[:NBUF]
        gsems = gsems_[:NBUF]
        ssems = ssems_[:NBUF]
        r0 = pl.multiple_of(s * RPT, 8)
        wbase = c * NGRP + s * gpt
        pltpu.sync_copy(zeros_hbm, acc.at[pl.ds(r0, RPT)])
        plsc.subcore_barrier()

        for h in range(nh):
            hb = wbase + h * HGRP
            pltpu.sync_copy(idx_hbm.at[pl.ds(hb, HGRP)], ibv)

            def step(j, carry):
                for b in range(NBUF):
                    g = j * NBUF + b

                    @pl.when(j > 0)
                    def _():
                        pltpu.make_async_copy(
                            rbs[b], acc.at[ibv.at[g - NBUF, 1]],
                            ssems[b]).wait()

                    pltpu.async_copy(y_hbm.at[ibv.at[g, 0]], rbs[b],
                                     gsems[b])
                for b in range(NBUF):
                    g = j * NBUF + b
                    pltpu.make_async_copy(
                        y_hbm.at[ibv.at[g, 0]], rbs[b], gsems[b]).wait()
                    pltpu.async_copy(rbs[b], acc.at[ibv.at[g, 1]], ssems[b],
                                     add=True)
                return carry

            lax.fori_loop(0, HGRP // NBUF, step, 0)
            for b in range(NBUF):
                g = HGRP - NBUF + b
                pltpu.make_async_copy(
                    rbs[b], acc.at[ibv.at[g, 1]], ssems[b]).wait()
        plsc.subcore_barrier()
        pltpu.sync_copy(acc.at[pl.ds(r0, RPT)], out_hbm.at[c, pl.ds(r0, RPT)])

    return body


@functools.cache
def _make_agg_call(d):
    return pl.kernel(
        _make_agg_body(d),
        out_type=jax.ShapeDtypeStruct((NC, NROWS, d), jnp.float32),
        mesh=plsc.VectorSubcoreMesh(core_axis_name="c", subcore_axis_name="s"),
        scratch_types=[
            pltpu.VMEM((HGRP, 2, GROUP), jnp.int32),
        ] + [pltpu.VMEM((GROUP, d), jnp.float32)] * NBUF + [
            pltpu.VMEM_SHARED((NROWS, d), jnp.float32),
        ] + [pltpu.SemaphoreType.DMA] * (2 * NBUF),
    )


def _agg256_call():
    return _make_agg_call(128)


def _dinv_of(deg_blk):
    deg = deg_blk[0, :, 0:1] + deg_blk[1, :, 0:1] + 1.0
    return lax.rsqrt(deg)


def _tc1a_body(x_ref, w_ref, y_ref):
    y_ref[...] = jnp.dot(x_ref[...], w_ref[...],
                         preferred_element_type=jnp.float32)


def _tc1b_body(xw_ref, deg_ref, y_ref):
    y_ref[...] = xw_ref[...] * _dinv_of(deg_ref[...])


def _tc_mid_body(a_ref, y_ref, deg_ref, b_ref, w_ref, o_ref):
    dinv = _dinv_of(deg_ref[...])
    a_full = jnp.concatenate([a_ref[0], a_ref[1]], axis=1) + y_ref[...]
    h = jnp.maximum(a_full * dinv + b_ref[...], 0.0)
    o_ref[...] = jnp.dot(h, w_ref[...],
                         preferred_element_type=jnp.float32) * dinv


def _tc3_body(a_ref, y_ref, deg_ref, b_ref, o_ref):
    dinv = _dinv_of(deg_ref[...])
    a_full = jnp.concatenate([a_ref[0], a_ref[1]], axis=1) + y_ref[...]
    o_ref[...] = jnp.maximum(a_full * dinv + b_ref[...], 0.0) * dinv


def _tc4_body(a_ref, y_ref, deg_ref, w3_ref, b3_ref, batch_ref,
              wf1_ref, bf1_ref, wf2_ref, bf2_ref, o_ref, s_acc, c_acc):
    j = pl.program_id(0)
    dinv = _dinv_of(deg_ref[...])
    z = (jnp.concatenate([a_ref[0], a_ref[1]], axis=1) + y_ref[...]) * dinv
    h = jnp.maximum(
        jnp.dot(z, w3_ref[...], preferred_element_type=jnp.float32)
        + b3_ref[...], 0.0)
    bvec = batch_ref[0]
    seg = lax.broadcasted_iota(jnp.int32, (G, BN), 0)
    mask = (seg == bvec).astype(jnp.float32)
    ps = jnp.dot(mask, h, preferred_element_type=jnp.float32)
    pc = jnp.broadcast_to(jnp.sum(mask, axis=1, keepdims=True), (G, 128))

    @pl.when(j == 0)
    def _():
        s_acc[...] = ps
        c_acc[...] = pc

    @pl.when(j > 0)
    def _():
        s_acc[...] += ps
        c_acc[...] += pc

    @pl.when(j == NBLK - 1)
    def _():
        pooled = s_acc[...] / jnp.maximum(c_acc[...][:, 0:1], 1.0)
        z = jnp.dot(pooled, wf1_ref[...],
                    preferred_element_type=jnp.float32) + bf1_ref[...]
        z = jnp.maximum(z, 0.0)
        z = jnp.dot(z, wf2_ref[...],
                    preferred_element_type=jnp.float32) + bf2_ref[...]
        o_ref[...] = 1.0 / (1.0 + jnp.exp(-z))


def _tc1a_call(x, w1):
    return pl.pallas_call(
        _tc1a_body,
        grid=(NBLK,),
        in_specs=[
            pl.BlockSpec((BN, F_IN), lambda j: (j, 0)),
            pl.BlockSpec((F_IN, H), lambda j: (0, 0)),
        ],
        out_specs=pl.BlockSpec((BN, H), lambda j: (j, 0)),
        out_shape=jax.ShapeDtypeStruct((N, H), jnp.float32),
    )(x, w1)


def _tc1b_call(xw, deg2):
    return pl.pallas_call(
        _tc1b_body,
        grid=(NBLK,),
        in_specs=[
            pl.BlockSpec((BN, H), lambda j: (j, 0)),
            pl.BlockSpec((NC, BN, 128), lambda j: (0, j, 0)),
        ],
        out_specs=pl.BlockSpec((BN, H), lambda j: (j, 0)),
        out_shape=jax.ShapeDtypeStruct((N, H), jnp.float32),
    )(xw, deg2)


def _tc_mid_call(agg, y, deg2, b, w):
    dout = w.shape[1]
    return pl.pallas_call(
        _tc_mid_body,
        grid=(NBLK,),
        in_specs=[
            pl.BlockSpec((NC, BN, 128), lambda j: (0, j, 0)),
            pl.BlockSpec((BN, H), lambda j: (j, 0)),
            pl.BlockSpec((NC, BN, 128), lambda j: (0, j, 0)),
            pl.BlockSpec((1, H), lambda j: (0, 0)),
            pl.BlockSpec((H, dout), lambda j: (0, 0)),
        ],
        out_specs=pl.BlockSpec((BN, dout), lambda j: (j, 0)),
        out_shape=jax.ShapeDtypeStruct((N, dout), jnp.float32),
    )(agg, y, deg2, b, w)


def _tc3_call(agg, y, deg2, b):
    return pl.pallas_call(
        _tc3_body,
        grid=(NBLK,),
        in_specs=[
            pl.BlockSpec((NC, BN, 128), lambda j: (0, j, 0)),
            pl.BlockSpec((BN, H), lambda j: (j, 0)),
            pl.BlockSpec((NC, BN, 128), lambda j: (0, j, 0)),
            pl.BlockSpec((1, H), lambda j: (0, 0)),
        ],
        out_specs=pl.BlockSpec((BN, H), lambda j: (j, 0)),
        out_shape=jax.ShapeDtypeStruct((N, H), jnp.float32),
    )(agg, y, deg2, b)


def _tc4_call(agg3, y3, deg2, w3, b3, batch3, wf1, bf1, wf2, bf2):
    return pl.pallas_call(
        _tc4_body,
        grid=(NBLK,),
        in_specs=[
            pl.BlockSpec((NC, BN, 128), lambda j: (0, j, 0)),
            pl.BlockSpec((BN, H), lambda j: (j, 0)),
            pl.BlockSpec((NC, BN, 128), lambda j: (0, j, 0)),
            pl.BlockSpec((H, OUT3), lambda j: (0, 0)),
            pl.BlockSpec((1, OUT3), lambda j: (0, 0)),
            pl.BlockSpec((1, 1, BN), lambda j: (j, 0, 0)),
            pl.BlockSpec((OUT3, 16), lambda j: (0, 0)),
            pl.BlockSpec((1, 16), lambda j: (0, 0)),
            pl.BlockSpec((16, 1), lambda j: (0, 0)),
            pl.BlockSpec((1, 1), lambda j: (0, 0)),
        ],
        out_specs=pl.BlockSpec((G, 1), lambda j: (0, 0)),
        out_shape=jax.ShapeDtypeStruct((G, 1), jnp.float32),
        scratch_shapes=[
            pltpu.VMEM((G, OUT3), jnp.float32),
            pltpu.VMEM((G, 128), jnp.float32),
        ],
    )(agg3, y3, deg2, w3, b3, batch3, wf1, bf1, wf2, bf2)


def kernel(x, edge_index, batch, W1, b1, W2, b2, W3, b3, Wf1, bf1, Wf2, bf2):
    src = edge_index[0]
    dst = edge_index[1]
    e = src.shape[0]
    pad = E_PAD - e
    src_p = jnp.concatenate([src, jnp.zeros((pad,), jnp.int32)])
    dst_p = jnp.concatenate([dst, jnp.full((pad,), N, jnp.int32)])
    src_g = src_p.reshape(NGRP, GROUP)
    dst_g = dst_p.reshape(NGRP, GROUP)
    idx256 = jnp.concatenate(
        [jnp.stack([src_g * 2, dst_g], axis=1),
         jnp.stack([src_g * 2 + 1, dst_g], axis=1)], axis=0)
    ones_col = (lax.broadcasted_iota(jnp.int32, (GROUP, 128), 1) == 0
                ).astype(jnp.float32)
    zeros128 = jnp.zeros((RPT, 128), jnp.float32)
    batch3 = batch.reshape(NBLK, 1, BN)

    deg2 = _deg_call()(dst_g, ones_col, zeros128)
    y1 = _tc1b_call(_tc1a_call(x, W1), deg2)
    agg1 = _agg256_call()(y1.reshape(2 * N, 128), idx256, zeros128)
    y2 = _tc_mid_call(agg1, y1, deg2, b1.reshape(1, H), W2)
    agg2 = _agg256_call()(y2.reshape(2 * N, 128), idx256, zeros128)
    y3 = _tc3_call(agg2, y2, deg2, b2.reshape(1, H))
    agg3 = _agg256_call()(y3.reshape(2 * N, 128), idx256, zeros128)
    return _tc4_call(agg3, y3, deg2, W3, b3.reshape(1, OUT3), batch3,
                     Wf1, bf1.reshape(1, 16), Wf2, bf2.reshape(1, 1))


# speedup vs baseline: 1.1455x; 1.1455x over previous
import functools

import jax
import jax.numpy as jnp
from jax import lax
from jax.experimental import pallas as pl
from jax.experimental.pallas import tpu as pltpu
from jax.experimental.pallas import tpu_sc as plsc

N = 10000
G = 64
F_IN = 256
H = 256
OUT3 = 32

NC = 2
NS = 16
GROUP = 64
NBUF = 4
DNBUF = 4
E_PAD = 163840
NGRP = E_PAD // GROUP
GPT = NGRP // NS
GPW = NGRP // (NC * NS)
HGRP = GPT // 4
NROWS = 10112
RPT = NROWS // NS

BN = 400
NBLK = N // BN


def _deg_body(dst_hbm, ones_hbm, zeros_hbm, out_hbm, dstv, onesv,
              acc, s0, s1, s2, s3):
    c = lax.axis_index("c")
    s = lax.axis_index("s")
    w = c * NS + s
    sems = [s0, s1, s2, s3]
    r0 = pl.multiple_of(s * RPT, 8)
    pltpu.sync_copy(dst_hbm.at[pl.ds(w * GPW, GPW)], dstv)
    pltpu.sync_copy(ones_hbm, onesv)
    pltpu.sync_copy(zeros_hbm, acc.at[pl.ds(r0, RPT)])
    plsc.subcore_barrier()

    def step(j, carry):
        for b in range(DNBUF):
            g = j * DNBUF + b

            @pl.when(j > 0)
            def _():
                pltpu.make_async_copy(
                    onesv, acc.at[dstv.at[g - DNBUF]], sems[b]).wait()

            pltpu.async_copy(onesv, acc.at[dstv.at[g]], sems[b], add=True)
        return carry

    lax.fori_loop(0, GPW // DNBUF, step, 0)
    for b in range(DNBUF):
        g = GPW - DNBUF + b
        pltpu.make_async_copy(onesv, acc.at[dstv.at[g]], sems[b]).wait()
    plsc.subcore_barrier()
    pltpu.sync_copy(acc.at[pl.ds(r0, RPT)], out_hbm.at[c, pl.ds(r0, RPT)])


@functools.cache
def _deg_call():
    return pl.kernel(
        _deg_body,
        out_type=jax.ShapeDtypeStruct((NC, NROWS, 128), jnp.float32),
        mesh=plsc.VectorSubcoreMesh(core_axis_name="c", subcore_axis_name="s"),
        scratch_types=[
            pltpu.VMEM((GPW, GROUP), jnp.int32),
            pltpu.VMEM((GROUP, 128), jnp.float32),
            pltpu.VMEM_SHARED((NROWS, 128), jnp.float32),
        ] + [pltpu.SemaphoreType.DMA] * DNBUF,
    )


def _make_agg_body(d, edge_split=False):
    gpt = GPW if edge_split else GPT
    nh = gpt // HGRP

    def body(y_hbm, idx_hbm, zeros_hbm, out_hbm, ibv, *bufs):
        rbs_ = list(bufs[:NBUF])
        acc = bufs[NBUF]
        gsems_ = list(bufs[NBUF + 1:NBUF + 1 + NBUF])
        ssems_ = list(bufs[NBUF + 1 + NBUF:])
        c = lax.axis_index("c")
        s = lax.axis_index("s")
        rbs = rbs_[:NBUF]
        gsems = gsems_[:NBUF]
        ssems = ssems_[:NBUF]
        r0 = pl.multiple_of(s * RPT, 8)
        if edge_split:
            wbase = (c * NS + s) * gpt
        else:
            wbase = c * NGRP + s * gpt
        pltpu.sync_copy(zeros_hbm, acc.at[pl.ds(r0, RPT)])
        plsc.subcore_barrier()

        for h in range(nh):
            hb = wbase + h * HGRP
            pltpu.sync_copy(idx_hbm.at[pl.ds(hb, HGRP)], ibv)

            def step(j, carry):
                for b in range(NBUF):
                    g = j * NBUF + b

                    @pl.when(j > 0)
                    def _():
                        pltpu.make_async_copy(
                            rbs[b], acc.at[ibv.at[g - NBUF, 1]],
                            ssems[b]).wait()

                    pltpu.async_copy(y_hbm.at[ibv.at[g, 0]], rbs[b],
                                     gsems[b])
                for b in range(NBUF):
                    g = j * NBUF + b
                    pltpu.make_async_copy(
                        y_hbm.at[ibv.at[g, 0]], rbs[b], gsems[b]).wait()
                    pltpu.async_copy(rbs[b], acc.at[ibv.at[g, 1]], ssems[b],
                                     add=True)
                return carry

            lax.fori_loop(0, HGRP // NBUF, step, 0)
            for b in range(NBUF):
                g = HGRP - NBUF + b
                pltpu.make_async_copy(
                    rbs[b], acc.at[ibv.at[g, 1]], ssems[b]).wait()
        plsc.subcore_barrier()
        pltpu.sync_copy(acc.at[pl.ds(r0, RPT)], out_hbm.at[c, pl.ds(r0, RPT)])

    return body


@functools.cache
def _make_agg_call(d, edge_split=False):
    return pl.kernel(
        _make_agg_body(d, edge_split),
        out_type=jax.ShapeDtypeStruct((NC, NROWS, d), jnp.float32),
        mesh=plsc.VectorSubcoreMesh(core_axis_name="c", subcore_axis_name="s"),
        scratch_types=[
            pltpu.VMEM((HGRP, 2, GROUP), jnp.int32),
        ] + [pltpu.VMEM((GROUP, d), jnp.float32)] * NBUF + [
            pltpu.VMEM_SHARED((NROWS, d), jnp.float32),
        ] + [pltpu.SemaphoreType.DMA] * (2 * NBUF),
    )


def _agg256_call():
    return _make_agg_call(128)


def _agg_es_call():
    return _make_agg_call(128, True)


def _dinv_of(deg_blk):
    deg = deg_blk[0, :, 0:1] + deg_blk[1, :, 0:1] + 1.0
    return lax.rsqrt(deg)


def _tc1_body(x_ref, deg_ref, w_ref, y_ref):
    dinv = _dinv_of(deg_ref[...])
    xw = jnp.dot(x_ref[...], w_ref[...], preferred_element_type=jnp.float32)
    y_ref[...] = xw * dinv


def _tc_mid_body(a_ref, y_ref, deg_ref, b_ref, w_ref, o_ref):
    dinv = _dinv_of(deg_ref[...])
    a_full = jnp.concatenate([a_ref[0], a_ref[1]], axis=1) + y_ref[...]
    h = jnp.maximum(a_full * dinv + b_ref[...], 0.0)
    o_ref[...] = jnp.dot(h, w_ref[...],
                         preferred_element_type=jnp.float32) * dinv


def _tc4_body(a_ref, y_ref, deg_ref, b3_ref, batch_ref,
              wf1_ref, bf1_ref, wf2_ref, bf2_ref, o_ref, s_acc, c_acc):
    j = pl.program_id(0)
    dinv = _dinv_of(deg_ref[...])
    z = (a_ref[0] + a_ref[1] + y_ref[...]) * dinv
    h = jnp.maximum(z[:, :OUT3] + b3_ref[...], 0.0)
    bvec = batch_ref[0]
    seg = lax.broadcasted_iota(jnp.int32, (G, BN), 0)
    mask = (seg == bvec).astype(jnp.float32)
    ps = jnp.dot(mask, h, preferred_element_type=jnp.float32)
    pc = jnp.broadcast_to(jnp.sum(mask, axis=1, keepdims=True), (G, 128))

    @pl.when(j == 0)
    def _():
        s_acc[...] = ps
        c_acc[...] = pc

    @pl.when(j > 0)
    def _():
        s_acc[...] += ps
        c_acc[...] += pc

    @pl.when(j == NBLK - 1)
    def _():
        pooled = s_acc[...] / jnp.maximum(c_acc[...][:, 0:1], 1.0)
        z = jnp.dot(pooled, wf1_ref[...],
                    preferred_element_type=jnp.float32) + bf1_ref[...]
        z = jnp.maximum(z, 0.0)
        z = jnp.dot(z, wf2_ref[...],
                    preferred_element_type=jnp.float32) + bf2_ref[...]
        o_ref[...] = 1.0 / (1.0 + jnp.exp(-z))


def _tc1_call(x, deg2, w1):
    return pl.pallas_call(
        _tc1_body,
        grid=(NBLK,),
        in_specs=[
            pl.BlockSpec((BN, F_IN), lambda j: (j, 0)),
            pl.BlockSpec((NC, BN, 128), lambda j: (0, j, 0)),
            pl.BlockSpec((F_IN, H), lambda j: (0, 0)),
        ],
        out_specs=pl.BlockSpec((BN, H), lambda j: (j, 0)),
        out_shape=jax.ShapeDtypeStruct((N, H), jnp.float32),
    )(x, deg2, w1)


def _tc_mid_call(agg, y, deg2, b, w):
    dout = w.shape[1]
    return pl.pallas_call(
        _tc_mid_body,
        grid=(NBLK,),
        in_specs=[
            pl.BlockSpec((NC, BN, 128), lambda j: (0, j, 0)),
            pl.BlockSpec((BN, H), lambda j: (j, 0)),
            pl.BlockSpec((NC, BN, 128), lambda j: (0, j, 0)),
            pl.BlockSpec((1, H), lambda j: (0, 0)),
            pl.BlockSpec((H, dout), lambda j: (0, 0)),
        ],
        out_specs=pl.BlockSpec((BN, dout), lambda j: (j, 0)),
        out_shape=jax.ShapeDtypeStruct((N, dout), jnp.float32),
    )(agg, y, deg2, b, w)


def _tc4_call(agg3, y3, deg2, b3, batch3, wf1, bf1, wf2, bf2):
    return pl.pallas_call(
        _tc4_body,
        grid=(NBLK,),
        in_specs=[
            pl.BlockSpec((NC, BN, 128), lambda j: (0, j, 0)),
            pl.BlockSpec((BN, 128), lambda j: (j, 0)),
            pl.BlockSpec((NC, BN, 128), lambda j: (0, j, 0)),
            pl.BlockSpec((1, OUT3), lambda j: (0, 0)),
            pl.BlockSpec((1, 1, BN), lambda j: (j, 0, 0)),
            pl.BlockSpec((OUT3, 16), lambda j: (0, 0)),
            pl.BlockSpec((1, 16), lambda j: (0, 0)),
            pl.BlockSpec((16, 1), lambda j: (0, 0)),
            pl.BlockSpec((1, 1), lambda j: (0, 0)),
        ],
        out_specs=pl.BlockSpec((G, 1), lambda j: (0, 0)),
        out_shape=jax.ShapeDtypeStruct((G, 1), jnp.float32),
        scratch_shapes=[
            pltpu.VMEM((G, OUT3), jnp.float32),
            pltpu.VMEM((G, 128), jnp.float32),
        ],
    )(agg3, y3, deg2, b3, batch3, wf1, bf1, wf2, bf2)


def kernel(x, edge_index, batch, W1, b1, W2, b2, W3, b3, Wf1, bf1, Wf2, bf2):
    src = edge_index[0]
    dst = edge_index[1]
    e = src.shape[0]
    pad = E_PAD - e
    src_p = jnp.concatenate([src, jnp.zeros((pad,), jnp.int32)])
    dst_p = jnp.concatenate([dst, jnp.full((pad,), N, jnp.int32)])
    src_g = src_p.reshape(NGRP, GROUP)
    dst_g = dst_p.reshape(NGRP, GROUP)
    idx_es = jnp.stack([src_g, dst_g], axis=1)
    idx256 = jnp.concatenate(
        [jnp.stack([src_g * 2, dst_g], axis=1),
         jnp.stack([src_g * 2 + 1, dst_g], axis=1)], axis=0)
    ones_col = (lax.broadcasted_iota(jnp.int32, (GROUP, 128), 1) == 0
                ).astype(jnp.float32)
    zeros128 = jnp.zeros((RPT, 128), jnp.float32)
    batch3 = batch.reshape(NBLK, 1, BN)

    deg2 = _deg_call()(dst_g, ones_col, zeros128)
    y1 = _tc1_call(x, deg2, W1)
    agg1 = _agg256_call()(y1.reshape(2 * N, 128), idx256, zeros128)
    y2 = _tc_mid_call(agg1, y1, deg2, b1.reshape(1, H), W2)
    agg2 = _agg256_call()(y2.reshape(2 * N, 128), idx256, zeros128)
    w3p = jnp.pad(W3, ((0, 0), (0, 128 - OUT3)))
    y3 = _tc_mid_call(agg2, y2, deg2, b2.reshape(1, H), w3p)
    agg3 = _agg_es_call()(y3, idx_es, zeros128)
    return _tc4_call(agg3, y3, deg2, b3.reshape(1, OUT3), batch3,
                     Wf1, bf1.reshape(1, 16), Wf2, bf2.reshape(1, 1))
